# baseline (speedup 1.0000x reference)
import jax
import jax.numpy as jnp
from jax.experimental import pallas as pl
from jax.experimental.pallas import tpu as pltpu

NUM_EDGES = 320000
IN_DIM = 128
OUT_DIM = 128

_ROWS = 1280


def _matmul_body(x_ref, w_ref, b_ref, o_ref):
    o_ref[...] = jnp.dot(x_ref[...], w_ref[...],
                         preferred_element_type=jnp.float32) + b_ref[...]


def _matmul(h_edges, W, b):
    n = h_edges.shape[0]
    grid = n // _ROWS
    return pl.pallas_call(
        _matmul_body,
        grid=(grid,),
        in_specs=[
            pl.BlockSpec((_ROWS, IN_DIM), lambda i: (i, 0)),
            pl.BlockSpec((IN_DIM, 3 * OUT_DIM), lambda i: (0, 0)),
            pl.BlockSpec((1, 3 * OUT_DIM), lambda i: (0, 0)),
        ],
        out_specs=pl.BlockSpec((_ROWS, 3 * OUT_DIM), lambda i: (i, 0)),
        out_shape=jax.ShapeDtypeStruct((n, 3 * OUT_DIM), jnp.float32),
    )(h_edges, W, b.reshape(1, -1))


def kernel(h_edges, edge_laplacian_lower_idxs, edge_laplacian_lower_weights,
           edge_laplacian_upper_idxs, edge_laplacian_upper_weights, W, b):
    h = _matmul(h_edges, W, b)
    h_lower = h[:, :OUT_DIM]
    h_intra = h[:, OUT_DIM:2 * OUT_DIM]
    h_upper = h[:, 2 * OUT_DIM:]
    msg_l = h_lower[edge_laplacian_lower_idxs[0]] * edge_laplacian_lower_weights[:, None]
    conv_l = jax.ops.segment_sum(msg_l, edge_laplacian_lower_idxs[1], num_segments=NUM_EDGES)
    msg_u = h_upper[edge_laplacian_upper_idxs[0]] * edge_laplacian_upper_weights[:, None]
    conv_u = jax.ops.segment_sum(msg_u, edge_laplacian_upper_idxs[1], num_segments=NUM_EDGES)
    return jnp.tanh(conv_l + h_intra + conv_u)
